# Initial kernel scaffold; baseline (speedup 1.0000x reference)
import jax
import jax.numpy as jnp
from jax.experimental import pallas as pl


def kernel(x, W_router, b_router, W_experts, b_experts, W_out, b_out):
    raise NotImplementedError("write your pallas kernel here")


# speedup vs baseline: 1.0938x; 1.0938x over previous
import functools

import jax
import jax.numpy as jnp
from jax.experimental import pallas as pl

_TILE = 512


def _body(x_ref, wr_ref, br_ref, w0_ref, w1_ref, wout_ref, b0_ref, b1_ref,
          bout_ref, out_ref):
    xt = x_ref[:]
    logits = jnp.dot(xt, wr_ref[:],
                     preferred_element_type=jnp.float32) + br_ref[:]

    T, E = logits.shape
    iota = jax.lax.broadcasted_iota(jnp.int32, (T, E), 1)

    m1 = jnp.max(logits, axis=1, keepdims=True)
    i1 = jnp.min(jnp.where(logits == m1, iota, E), axis=1, keepdims=True)
    rest = jnp.where(iota == i1, -jnp.inf, logits)
    m2 = jnp.max(rest, axis=1, keepdims=True)
    i2 = jnp.min(jnp.where(rest == m2, iota, E), axis=1, keepdims=True)

    mask0 = ((i1 == 0) | (i2 == 0)).astype(jnp.float32)
    mask1 = ((i1 == 1) | (i2 == 1)).astype(jnp.float32)
    p1 = jax.nn.sigmoid(m1 - m2)
    p2 = 1.0 - p1

    z0 = jnp.dot(xt, w0_ref[:], preferred_element_type=jnp.float32)
    h0 = jax.nn.gelu(mask0 * z0 + b0_ref[:], approximate=True)
    z1 = jnp.dot(xt, w1_ref[:], preferred_element_type=jnp.float32)
    h1 = jax.nn.gelu(mask1 * z1 + b1_ref[:], approximate=True)

    combined = p1 * h0 + p2 * h1
    out_ref[:] = jnp.dot(combined, wout_ref[:],
                         preferred_element_type=jnp.float32) + bout_ref[:]


@functools.partial(jax.jit, static_argnames=())
def kernel(x, W_router, b_router, W_experts, b_experts, W_out, b_out):
    B, S, D = x.shape
    E = W_router.shape[1]
    F = W_out.shape[1]
    N = B * S
    xf = x.reshape(N, D)
    w0 = W_experts[0]
    w1 = W_experts[1]
    b0 = b_experts[0].reshape(1, -1)
    b1 = b_experts[1].reshape(1, -1)
    br = b_router.reshape(1, E)
    bo = b_out.reshape(1, F)

    grid = (N // _TILE,)
    const = lambda i: (0, 0)
    out = pl.pallas_call(
        _body,
        grid=grid,
        in_specs=[
            pl.BlockSpec((_TILE, D), lambda i: (i, 0)),
            pl.BlockSpec((D, E), const),
            pl.BlockSpec((1, E), const),
            pl.BlockSpec((D, D), const),
            pl.BlockSpec((D, D), const),
            pl.BlockSpec((D, F), const),
            pl.BlockSpec((1, D), const),
            pl.BlockSpec((1, D), const),
            pl.BlockSpec((1, F), const),
        ],
        out_specs=pl.BlockSpec((_TILE, F), lambda i: (i, 0)),
        out_shape=jax.ShapeDtypeStruct((N, F), jnp.float32),
    )(xf, W_router, br, w0, w1, W_out, b0, b1, bo)
    return out.reshape(B, S, F)
